# baseline (speedup 1.0000x reference)
import functools
import jax
import jax.numpy as jnp
from jax import lax
from jax.experimental import pallas as pl
from jax.experimental.pallas import tpu as pltpu
from jax.experimental.pallas import tpu_sc as plsc

B = 16384
H = 64

_info = plsc.get_sparse_core_info()
NC = _info.num_cores
NS = _info.num_subcores
L = _info.num_lanes
NW = NC * NS
RPW = B // NW
IC = 128
NIC = RPW // IC

_mesh = plsc.VectorSubcoreMesh(core_axis_name="c", subcore_axis_name="s")


@functools.partial(
    pl.kernel,
    mesh=_mesh,
    out_type=jax.ShapeDtypeStruct((B,), jnp.float32),
    compiler_params=pltpu.CompilerParams(needs_layout_passes=False,
                                         use_tc_tiling_on_sc=False),
    scratch_types=[
        pltpu.VMEM((NIC, IC), jnp.int32),
        pltpu.VMEM((NIC, IC), jnp.int32),
        pltpu.VMEM((RPW,), jnp.float32),
        pltpu.VMEM((RPW,), jnp.float32),
        pltpu.VMEM((RPW, H), jnp.float32),
        pltpu.VMEM((RPW, H), jnp.float32),
        pltpu.VMEM((H,), jnp.float32),
        pltpu.VMEM((L,), jnp.float32),
        pltpu.VMEM((L, L), jnp.float32),
        pltpu.VMEM((RPW,), jnp.float32),
        pltpu.SemaphoreType.DMA,
        pltpu.SemaphoreType.DMA,
    ],
)
def _lf_kernel(uf_hbm, uid_hbm, if_hbm, iid_hbm, w_hbm, b_hbm, bu_hbm, bi_hbm,
               out_hbm, uidx_v, iidx_v, ub_v, ib_v, uf_v, if_v, w_v, b_v,
               t_v, out_v, sem_feat, sem_g):
    wid = lax.axis_index("s") * NC + lax.axis_index("c")
    row0 = wid * RPW
    ic0 = wid * NIC

    pltpu.sync_copy(uid_hbm.at[pl.ds(ic0, NIC)], uidx_v)
    pltpu.sync_copy(iid_hbm.at[pl.ds(ic0, NIC)], iidx_v)

    cf1 = pltpu.async_copy(uf_hbm.at[pl.ds(row0, RPW)], uf_v, sem_feat)
    cf2 = pltpu.async_copy(if_hbm.at[pl.ds(row0, RPW)], if_v, sem_feat)
    gathers = []
    for j in range(NIC):
        gathers.append(
            pltpu.async_copy(bu_hbm.at[uidx_v.at[j]],
                             ub_v.at[pl.ds(j * IC, IC)], sem_g))
        gathers.append(
            pltpu.async_copy(bi_hbm.at[iidx_v.at[j]],
                             ib_v.at[pl.ds(j * IC, IC)], sem_g))

    pltpu.sync_copy(w_hbm, w_v)
    pltpu.sync_copy(b_hbm, b_v)

    for g in gathers:
        g.wait()
    cf1.wait()
    cf2.wait()

    ws = [w_v[pl.ds(k * L, L)] for k in range(H // L)]
    bv = b_v[...]
    rows = lax.iota(jnp.int32, L)

    def groupbody(g, carry):
        base = pl.multiple_of(g * L, L)
        for u in range(L):
            r = base + u
            a = uf_v[r, pl.ds(0, L)] * if_v[r, pl.ds(0, L)] * ws[0]
            for k in range(1, H // L):
                a = a + (uf_v[r, pl.ds(k * L, L)]
                         * if_v[r, pl.ds(k * L, L)] * ws[k])
            t_v[u, pl.ds(0, L)] = a
        s = plsc.load_gather(t_v, [rows, jnp.full((L,), 0, jnp.int32)])
        for l in range(1, L):
            s = s + plsc.load_gather(t_v, [rows, jnp.full((L,), l, jnp.int32)])
        out_v[pl.ds(base, L)] = (s + ub_v[pl.ds(base, L)]
                                 + ib_v[pl.ds(base, L)] + bv)
        return carry

    lax.fori_loop(0, RPW // L, groupbody, 0)

    pltpu.sync_copy(out_v, out_hbm.at[pl.ds(row0, RPW)])


def kernel(user_feature, user_id, item_feature, item_id, W, b, b_user, b_item):
    uid = user_id.reshape(B // IC, IC)
    iid = item_id.reshape(B // IC, IC)
    w = W.reshape(H)
    bvec = jnp.broadcast_to(b, (L,))
    out = _lf_kernel(user_feature, uid, item_feature, iid, w, bvec,
                     b_user, b_item)
    return out.reshape(B, 1)


# speedup vs baseline: 1.7987x; 1.7987x over previous
import functools
import jax
import jax.numpy as jnp
from jax import lax
from jax.experimental import pallas as pl
from jax.experimental.pallas import tpu as pltpu
from jax.experimental.pallas import tpu_sc as plsc

B = 16384
H = 64

_info = plsc.get_sparse_core_info()
NC = _info.num_cores
NS = _info.num_subcores
L = _info.num_lanes
NW = NC * NS
RPW = B // NW
IC = 128
NIC = RPW // IC
GB = 2

_mesh = plsc.VectorSubcoreMesh(core_axis_name="c", subcore_axis_name="s")


@functools.partial(
    pl.kernel,
    mesh=_mesh,
    out_type=jax.ShapeDtypeStruct((B,), jnp.float32),
    compiler_params=pltpu.CompilerParams(needs_layout_passes=False,
                                         use_tc_tiling_on_sc=True),
    scratch_types=[
        pltpu.VMEM((NIC, IC), jnp.int32),
        pltpu.VMEM((NIC, IC), jnp.int32),
        pltpu.VMEM((RPW,), jnp.float32),
        pltpu.VMEM((RPW,), jnp.float32),
        pltpu.VMEM((H, RPW), jnp.float32),
        pltpu.VMEM((H, RPW), jnp.float32),
        pltpu.VMEM((H, L), jnp.float32),
        pltpu.VMEM((L,), jnp.float32),
        pltpu.VMEM((RPW,), jnp.float32),
        pltpu.SemaphoreType.DMA,
        pltpu.SemaphoreType.DMA,
    ],
)
def _lf_kernel(uf_hbm, uid_hbm, if_hbm, iid_hbm, w_hbm, b_hbm, bu_hbm, bi_hbm,
               out_hbm, uidx_v, iidx_v, ub_v, ib_v, uf_v, if_v, w_v, b_v,
               out_v, sem_feat, sem_g):
    wid = lax.axis_index("s") * NC + lax.axis_index("c")
    col0 = wid * RPW
    ic0 = wid * NIC

    pltpu.sync_copy(uid_hbm.at[pl.ds(ic0, NIC)], uidx_v)
    pltpu.sync_copy(iid_hbm.at[pl.ds(ic0, NIC)], iidx_v)

    cf1 = pltpu.async_copy(uf_hbm.at[:, pl.ds(col0, RPW)], uf_v, sem_feat)
    cf2 = pltpu.async_copy(if_hbm.at[:, pl.ds(col0, RPW)], if_v, sem_feat)
    gathers = []
    for j in range(NIC):
        gathers.append(
            pltpu.async_copy(bu_hbm.at[uidx_v.at[j]],
                             ub_v.at[pl.ds(j * IC, IC)], sem_g))
        gathers.append(
            pltpu.async_copy(bi_hbm.at[iidx_v.at[j]],
                             ib_v.at[pl.ds(j * IC, IC)], sem_g))

    pltpu.sync_copy(w_hbm, w_v)
    pltpu.sync_copy(b_hbm, b_v)

    for g in gathers:
        g.wait()
    cf1.wait()
    cf2.wait()

    bv = b_v[...]

    def groupbody(gg, carry):
        base = pl.multiple_of(gg * (GB * L), GB * L)
        accs = [None] * GB
        for h in range(H):
            wbh = w_v[h, pl.ds(0, L)]
            for q in range(GB):
                prod = (uf_v[h, pl.ds(base + q * L, L)]
                        * if_v[h, pl.ds(base + q * L, L)] * wbh)
                accs[q] = prod if h == 0 else accs[q] + prod
        for q in range(GB):
            o = pl.ds(base + q * L, L)
            out_v[o] = accs[q] + ub_v[o] + ib_v[o] + bv
        return carry

    lax.fori_loop(0, RPW // (GB * L), groupbody, 0)

    pltpu.sync_copy(out_v, out_hbm.at[pl.ds(col0, RPW)])


def kernel(user_feature, user_id, item_feature, item_id, W, b, b_user, b_item):
    uft = user_feature.T
    ift = item_feature.T
    uid = user_id.reshape(B // IC, IC)
    iid = item_id.reshape(B // IC, IC)
    wb = jnp.broadcast_to(W.reshape(H, 1), (H, L))
    bvec = jnp.broadcast_to(b, (L,))
    out = _lf_kernel(uft, uid, ift, iid, wb, bvec, b_user, b_item)
    return out.reshape(B, 1)
